# baseline (speedup 1.0000x reference)
import functools

import jax
import jax.numpy as jnp
from jax import lax
from jax.experimental import pallas as pl
from jax.experimental.pallas import tpu as pltpu
from jax.experimental.pallas import tpu_sc as plsc

_B = 4096
_D = 1024
_EPS = 1e-8
_L = 16
_NC = 2
_NS = 16
_NRW = 4
_NCW = 8
_CW = _D // _NCW
_GPER = 19
_PIECES = ((0, 5), (5, 5), (10, 5), (15, 4))
_PMAXR = 40

_LN2 = 0.6931471805599453
_SQRT2 = 1.4142135623730951

_REGIONS = (
    (1755, 2340, (1, 0)),
    (2340, 2925, (0, 2)),
    (2925, 3510, (1, 2)),
    (3510, 4096, (0, 1, 2)),
)

_INTERIOR = tuple((-(-rs // 8), re // 8) for (rs, re, _) in _REGIONS)

_TASKS = tuple((k, off, g)
               for k in range(len(_REGIONS))
               for (off, g) in _PIECES)

_HAS_R = (0, 3, 4, 6)
_HAS_D = (1, 3, 5, 6)
_HAS_T = (2, 4, 5, 6)
_HAS_P = (6,)


def _vlog(x):
    bits = lax.bitcast_convert_type(x, jnp.int32)
    e = (bits >> 23) - 127
    m = lax.bitcast_convert_type((bits & 0x007FFFFF) | 0x3F800000, jnp.float32)
    big = m > _SQRT2
    m = jnp.where(big, m * 0.5, m)
    ef = e.astype(jnp.float32) + jnp.where(big, 1.0, 0.0)
    f = m - 1.0
    q = (-0.5000041083608477
         + f * (0.3332492391225158
                + f * (-0.24932832776171132
                       + f * (0.20346370495399466
                              + f * (-0.18482372758788945
                                     + f * 0.12282081708318798)))))
    return ef * _LN2 + (f + (f * f) * q)


def _member(bid, regions):
    acc = jnp.float32(0.0)
    for rid in regions:
        acc = jnp.where(bid == rid, jnp.float32(1.0), acc)
    return acc


def _sc_body(mu_r, mu_d, mu_t, lv_r, lv_d, lv_t, o_mu, o_lv, *scr):
    bufs = (scr[0:8], scr[8:16])
    bbuf = scr[16:24]
    in_sems = scr[24:26]
    out_sems = scr[26:28]
    bin_sem = scr[28]
    bout_sem = scr[29]

    wid = lax.axis_index("s") * _NC + lax.axis_index("c")
    rw = wid >> 3
    cw = wid & 7
    col = cw * _CW
    mus = (mu_r, mu_d, mu_t)
    lvs = (lv_r, lv_d, lv_t)

    gbase = [gs + jnp.minimum(rw * _GPER, (ge - gs) - _GPER)
             for (gs, ge) in _INTERIOR]

    bid = rw + 2
    brow = (bid + 1) * 584
    bhs = []
    for j in range(3):
        bhs.append(pltpu.async_copy(
            mus[j].at[pl.ds(brow, 8), pl.ds(col, _CW)], bbuf[j], bin_sem))
        bhs.append(pltpu.async_copy(
            lvs[j].at[pl.ds(brow, 8), pl.ds(col, _CW)], bbuf[3 + j], bin_sem))

    def boundary_finish():
        m0, m1, m2, l0, l1, l2, omu, olv = bbuf
        for h in bhs:
            h.wait()
        wrA = _member(bid, _HAS_R)
        wdA = _member(bid, _HAS_D)
        wtA = _member(bid, _HAS_T)
        wpA = _member(bid, _HAS_P)
        wrB = _member(bid + 1, _HAS_R)
        wdB = _member(bid + 1, _HAS_D)
        wtB = _member(bid + 1, _HAS_T)
        wpB = _member(bid + 1, _HAS_P)
        cut = bid + 1

        @plsc.parallel_loop(0, 8 * (_CW // _L), unroll=5)
        def _(i):
            r = i >> 3
            c = (i & 7) << 4
            inA = r < cut
            wr = jnp.where(inA, wrA, wrB)
            wd = jnp.where(inA, wdA, wdB)
            wt = jnp.where(inA, wtA, wtB)
            wp = jnp.where(inA, wpA, wpB)
            mua = m0[r, pl.ds(c, _L)]
            mub = m1[r, pl.ds(c, _L)]
            muc = m2[r, pl.ds(c, _L)]
            va = jnp.exp(l0[r, pl.ds(c, _L)]) + _EPS
            vb = jnp.exp(l1[r, pl.ds(c, _L)]) + _EPS
            vc = jnp.exp(l2[r, pl.ds(c, _L)]) + _EPS
            ab = va * vb
            ac = va * vc
            bc = vb * vc
            abc = ab * vc
            rec = 1.0 / (wr * bc + wd * ac + wt * ab + wp * abc)
            omu[r, pl.ds(c, _L)] = (wr * mua * bc + wd * mub * ac
                                    + wt * muc * ab) * rec
            olv[r, pl.ds(c, _L)] = _vlog(abc * rec + _EPS)

        pltpu.async_copy(omu, o_mu.at[pl.ds(brow, 8), pl.ds(col, _CW)],
                         bout_sem).wait()
        pltpu.async_copy(olv, o_lv.at[pl.ds(brow, 8), pl.ds(col, _CW)],
                         bout_sem).wait()

    def start_in(ti):
        k, off, g = _TASKS[ti]
        mods = _REGIONS[k][2]
        s = ti % 2
        row = (gbase[k] + off) * 8
        R = g * 8
        hs = []
        for j, m in enumerate(mods):
            hs.append(pltpu.async_copy(
                mus[m].at[pl.ds(row, R), pl.ds(col, _CW)],
                bufs[s][j].at[pl.ds(0, R)], in_sems[s]))
            hs.append(pltpu.async_copy(
                lvs[m].at[pl.ds(row, R), pl.ds(col, _CW)],
                bufs[s][3 + j].at[pl.ds(0, R)], in_sems[s]))
        return hs

    def start_out(ti):
        k, off, g = _TASKS[ti]
        s = ti % 2
        row = (gbase[k] + off) * 8
        R = g * 8
        return [pltpu.async_copy(bufs[s][6].at[pl.ds(0, R)],
                                 o_mu.at[pl.ds(row, R), pl.ds(col, _CW)],
                                 out_sems[s]),
                pltpu.async_copy(bufs[s][7].at[pl.ds(0, R)],
                                 o_lv.at[pl.ds(row, R), pl.ds(col, _CW)],
                                 out_sems[s])]

    def compute(ti):
        k, off, g = _TASKS[ti]
        mods = _REGIONS[k][2]
        s = ti % 2
        m0, m1, m2, l0, l1, l2, omu, olv = bufs[s]
        nvec = g * 8 * (_CW // _L)
        if len(mods) == 2:
            @plsc.parallel_loop(0, nvec, unroll=5)
            def _(i):
                r = i >> 3
                c = (i & 7) << 4
                mua = m0[r, pl.ds(c, _L)]
                mub = m1[r, pl.ds(c, _L)]
                la = l0[r, pl.ds(c, _L)]
                lb = l1[r, pl.ds(c, _L)]
                va = jnp.exp(la) + _EPS
                vb = jnp.exp(lb) + _EPS
                den = va + vb
                rec = 1.0 / den
                omu[r, pl.ds(c, _L)] = (mua * vb + mub * va) * rec
                olv[r, pl.ds(c, _L)] = (la + lb) - _vlog(den)
        else:
            @plsc.parallel_loop(0, nvec, unroll=5)
            def _(i):
                r = i >> 3
                c = (i & 7) << 4
                mua = m0[r, pl.ds(c, _L)]
                mub = m1[r, pl.ds(c, _L)]
                muc = m2[r, pl.ds(c, _L)]
                la = l0[r, pl.ds(c, _L)]
                lb = l1[r, pl.ds(c, _L)]
                lc = l2[r, pl.ds(c, _L)]
                va = jnp.exp(la) + _EPS
                vb = jnp.exp(lb) + _EPS
                vc = jnp.exp(lc) + _EPS
                ab = va * vb
                ac = va * vc
                bc = vb * vc
                abc = ab * vc
                den = ab + ac + bc + abc
                rec = 1.0 / den
                omu[r, pl.ds(c, _L)] = (mua * bc + mub * ac + muc * ab) * rec
                olv[r, pl.ds(c, _L)] = ((la + lb) + lc) - _vlog(den)

    n = len(_TASKS)
    hout = [None] * n
    hin = start_in(0)
    for i in range(n):
        nxt = None
        if i + 1 < n:
            if i >= 1:
                for h in hout[i - 1]:
                    h.wait()
            nxt = start_in(i + 1)
        for h in hin:
            h.wait()
        compute(i)
        hout[i] = start_out(i)
        hin = nxt
    for h in hout[n - 2]:
        h.wait()
    for h in hout[n - 1]:
        h.wait()

    boundary_finish()


_fused = functools.partial(
    pl.kernel,
    out_type=(jax.ShapeDtypeStruct((_B, _D), jnp.float32),
              jax.ShapeDtypeStruct((_B, _D), jnp.float32)),
    mesh=plsc.VectorSubcoreMesh(core_axis_name="c", subcore_axis_name="s",
                                num_cores=_NC, num_subcores=_NS),
    scratch_types=([pltpu.VMEM((_PMAXR, _CW), jnp.float32)] * 16
                   + [pltpu.VMEM((8, _CW), jnp.float32)] * 8
                   + [pltpu.SemaphoreType.DMA] * 6),
)(_sc_body)


@jax.jit
def kernel(mu_rgb, mu_depth, mu_touch, logvar_rgb, logvar_depth, logvar_touch):
    o_mu, o_lv = _fused(mu_rgb, mu_depth, mu_touch,
                        logvar_rgb, logvar_depth, logvar_touch)
    pre_mu = jnp.concatenate(
        [mu_rgb[:585], mu_depth[585:1170], mu_touch[1170:1752]], axis=0)
    pre_lv = jnp.concatenate(
        [logvar_rgb[:585], logvar_depth[585:1170], logvar_touch[1170:1752]],
        axis=0)
    return (lax.dynamic_update_slice(o_mu, pre_mu, (0, 0)),
            lax.dynamic_update_slice(o_lv, pre_lv, (0, 0)))


# speedup vs baseline: 1.0047x; 1.0047x over previous
import functools

import jax
import jax.numpy as jnp
from jax import lax
from jax.experimental import pallas as pl
from jax.experimental.pallas import tpu as pltpu
from jax.experimental.pallas import tpu_sc as plsc

_B = 4096
_D = 1024
_EPS = 1e-8
_L = 16
_NC = 2
_NS = 16
_NRW = 4
_NCW = 8
_CW = _D // _NCW
_GPER = 19
_PIECES = ((0, 5), (5, 5), (10, 5), (15, 4))
_PMAXR = 40

_LN2 = 0.6931471805599453
_SQRT2 = 1.4142135623730951

_REGIONS = (
    (1755, 2340, (1, 0)),
    (2340, 2925, (0, 2)),
    (2925, 3510, (1, 2)),
    (3510, 4096, (0, 1, 2)),
)

_INTERIOR = tuple((-(-rs // 8), re // 8) for (rs, re, _) in _REGIONS)

_TASKS = tuple((k, off, g)
               for k in range(len(_REGIONS))
               for (off, g) in _PIECES)

_HAS_R = (0, 3, 4, 6)
_HAS_D = (1, 3, 5, 6)
_HAS_T = (2, 4, 5, 6)
_HAS_P = (6,)


def _vlog(x):
    bits = lax.bitcast_convert_type(x, jnp.int32)
    e = (bits >> 23) - 127
    m = lax.bitcast_convert_type((bits & 0x007FFFFF) | 0x3F800000, jnp.float32)
    big = m > _SQRT2
    m = jnp.where(big, m * 0.5, m)
    ef = e.astype(jnp.float32) + jnp.where(big, 1.0, 0.0)
    f = m - 1.0
    q = (-0.5000041083608477
         + f * (0.3332492391225158
                + f * (-0.24932832776171132
                       + f * (0.20346370495399466
                              + f * (-0.18482372758788945
                                     + f * 0.12282081708318798)))))
    return ef * _LN2 + (f + (f * f) * q)


def _member(bid, regions):
    acc = jnp.float32(0.0)
    for rid in regions:
        acc = jnp.where(bid == rid, jnp.float32(1.0), acc)
    return acc


def _sc_body(mu_r, mu_d, mu_t, lv_r, lv_d, lv_t, o_mu, o_lv, *scr):
    bufs = (scr[0:8], scr[8:16])
    bbuf = scr[16:24]
    in_sems = scr[24:26]
    out_sems = scr[26:28]
    bin_sem = scr[28]
    bout_sem = scr[29]

    wid = lax.axis_index("s") * _NC + lax.axis_index("c")
    rw = wid >> 3
    cw = wid & 7
    col = cw * _CW
    mus = (mu_r, mu_d, mu_t)
    lvs = (lv_r, lv_d, lv_t)

    gbase = [gs + jnp.minimum(rw * _GPER, (ge - gs) - _GPER)
             for (gs, ge) in _INTERIOR]

    bid = rw + 2
    brow = (bid + 1) * 584
    bhs = []
    for j in range(3):
        bhs.append(pltpu.async_copy(
            mus[j].at[pl.ds(brow, 8), pl.ds(col, _CW)], bbuf[j], bin_sem))
        bhs.append(pltpu.async_copy(
            lvs[j].at[pl.ds(brow, 8), pl.ds(col, _CW)], bbuf[3 + j], bin_sem))

    def boundary_finish():
        m0, m1, m2, l0, l1, l2, omu, olv = bbuf
        for h in bhs:
            h.wait()
        wrA = _member(bid, _HAS_R)
        wdA = _member(bid, _HAS_D)
        wtA = _member(bid, _HAS_T)
        wpA = _member(bid, _HAS_P)
        wrB = _member(bid + 1, _HAS_R)
        wdB = _member(bid + 1, _HAS_D)
        wtB = _member(bid + 1, _HAS_T)
        wpB = _member(bid + 1, _HAS_P)
        cut = bid + 1

        @plsc.parallel_loop(0, 8 * (_CW // _L), unroll=4)
        def _(i):
            r = i >> 3
            c = (i & 7) << 4
            inA = r < cut
            wr = jnp.where(inA, wrA, wrB)
            wd = jnp.where(inA, wdA, wdB)
            wt = jnp.where(inA, wtA, wtB)
            wp = jnp.where(inA, wpA, wpB)
            mua = m0[r, pl.ds(c, _L)]
            mub = m1[r, pl.ds(c, _L)]
            muc = m2[r, pl.ds(c, _L)]
            va = jnp.exp(l0[r, pl.ds(c, _L)]) + _EPS
            vb = jnp.exp(l1[r, pl.ds(c, _L)]) + _EPS
            vc = jnp.exp(l2[r, pl.ds(c, _L)]) + _EPS
            ab = va * vb
            ac = va * vc
            bc = vb * vc
            abc = ab * vc
            rec = 1.0 / (wr * bc + wd * ac + wt * ab + wp * abc)
            omu[r, pl.ds(c, _L)] = (wr * mua * bc + wd * mub * ac
                                    + wt * muc * ab) * rec
            olv[r, pl.ds(c, _L)] = _vlog(abc * rec + _EPS)

        pltpu.async_copy(omu, o_mu.at[pl.ds(brow, 8), pl.ds(col, _CW)],
                         bout_sem).wait()
        pltpu.async_copy(olv, o_lv.at[pl.ds(brow, 8), pl.ds(col, _CW)],
                         bout_sem).wait()

    def start_in(ti):
        k, off, g = _TASKS[ti]
        mods = _REGIONS[k][2]
        s = ti % 2
        row = (gbase[k] + off) * 8
        R = g * 8
        hs = []
        for j, m in enumerate(mods):
            hs.append(pltpu.async_copy(
                mus[m].at[pl.ds(row, R), pl.ds(col, _CW)],
                bufs[s][j].at[pl.ds(0, R)], in_sems[s]))
            hs.append(pltpu.async_copy(
                lvs[m].at[pl.ds(row, R), pl.ds(col, _CW)],
                bufs[s][3 + j].at[pl.ds(0, R)], in_sems[s]))
        return hs

    def start_out(ti):
        k, off, g = _TASKS[ti]
        s = ti % 2
        row = (gbase[k] + off) * 8
        R = g * 8
        return [pltpu.async_copy(bufs[s][6].at[pl.ds(0, R)],
                                 o_mu.at[pl.ds(row, R), pl.ds(col, _CW)],
                                 out_sems[s]),
                pltpu.async_copy(bufs[s][7].at[pl.ds(0, R)],
                                 o_lv.at[pl.ds(row, R), pl.ds(col, _CW)],
                                 out_sems[s])]

    def compute(ti):
        k, off, g = _TASKS[ti]
        mods = _REGIONS[k][2]
        s = ti % 2
        m0, m1, m2, l0, l1, l2, omu, olv = bufs[s]
        nvec = g * 8 * (_CW // _L)
        if len(mods) == 2:
            @plsc.parallel_loop(0, nvec, unroll=4)
            def _(i):
                r = i >> 3
                c = (i & 7) << 4
                mua = m0[r, pl.ds(c, _L)]
                mub = m1[r, pl.ds(c, _L)]
                la = l0[r, pl.ds(c, _L)]
                lb = l1[r, pl.ds(c, _L)]
                va = jnp.exp(la) + _EPS
                vb = jnp.exp(lb) + _EPS
                den = va + vb
                rec = 1.0 / den
                omu[r, pl.ds(c, _L)] = (mua * vb + mub * va) * rec
                olv[r, pl.ds(c, _L)] = (la + lb) - _vlog(den)
        else:
            @plsc.parallel_loop(0, nvec, unroll=4)
            def _(i):
                r = i >> 3
                c = (i & 7) << 4
                mua = m0[r, pl.ds(c, _L)]
                mub = m1[r, pl.ds(c, _L)]
                muc = m2[r, pl.ds(c, _L)]
                la = l0[r, pl.ds(c, _L)]
                lb = l1[r, pl.ds(c, _L)]
                lc = l2[r, pl.ds(c, _L)]
                va = jnp.exp(la) + _EPS
                vb = jnp.exp(lb) + _EPS
                vc = jnp.exp(lc) + _EPS
                ab = va * vb
                ac = va * vc
                bc = vb * vc
                abc = ab * vc
                den = ab + ac + bc + abc
                rec = 1.0 / den
                omu[r, pl.ds(c, _L)] = (mua * bc + mub * ac + muc * ab) * rec
                olv[r, pl.ds(c, _L)] = ((la + lb) + lc) - _vlog(den)

    n = len(_TASKS)
    hout = [None] * n
    hin = start_in(0)
    for i in range(n):
        nxt = None
        if i + 1 < n:
            if i >= 1:
                for h in hout[i - 1]:
                    h.wait()
            nxt = start_in(i + 1)
        for h in hin:
            h.wait()
        compute(i)
        hout[i] = start_out(i)
        hin = nxt
    for h in hout[n - 2]:
        h.wait()
    for h in hout[n - 1]:
        h.wait()

    boundary_finish()


_fused = functools.partial(
    pl.kernel,
    out_type=(jax.ShapeDtypeStruct((_B, _D), jnp.float32),
              jax.ShapeDtypeStruct((_B, _D), jnp.float32)),
    mesh=plsc.VectorSubcoreMesh(core_axis_name="c", subcore_axis_name="s",
                                num_cores=_NC, num_subcores=_NS),
    scratch_types=([pltpu.VMEM((_PMAXR, _CW), jnp.float32)] * 16
                   + [pltpu.VMEM((8, _CW), jnp.float32)] * 8
                   + [pltpu.SemaphoreType.DMA] * 6),
)(_sc_body)


@jax.jit
def kernel(mu_rgb, mu_depth, mu_touch, logvar_rgb, logvar_depth, logvar_touch):
    o_mu, o_lv = _fused(mu_rgb, mu_depth, mu_touch,
                        logvar_rgb, logvar_depth, logvar_touch)
    pre_mu = jnp.concatenate(
        [mu_rgb[:585], mu_depth[585:1170], mu_touch[1170:1752]], axis=0)
    pre_lv = jnp.concatenate(
        [logvar_rgb[:585], logvar_depth[585:1170], logvar_touch[1170:1752]],
        axis=0)
    return (lax.dynamic_update_slice(o_mu, pre_mu, (0, 0)),
            lax.dynamic_update_slice(o_lv, pre_lv, (0, 0)))


# speedup vs baseline: 1.0087x; 1.0040x over previous
import functools

import jax
import jax.numpy as jnp
from jax import lax
from jax.experimental import pallas as pl
from jax.experimental.pallas import tpu as pltpu
from jax.experimental.pallas import tpu_sc as plsc

_B = 4096
_D = 1024
_EPS = 1e-8
_L = 16
_NC = 2
_NS = 16
_NRW = 4
_NCW = 8
_CW = _D // _NCW
_GPER = 19
_PIECES = ((0, 5), (5, 5), (10, 5), (15, 4))
_PMAXR = 40

_LN2 = 0.6931471805599453
_SQRT2 = 1.4142135623730951

_REGIONS = (
    (1755, 2340, (1, 0)),
    (2340, 2925, (0, 2)),
    (2925, 3510, (1, 2)),
    (3510, 4096, (0, 1, 2)),
)

_INTERIOR = tuple((-(-rs // 8), re // 8) for (rs, re, _) in _REGIONS)

_TASKS = tuple((k, off, g)
               for k in range(len(_REGIONS))
               for (off, g) in _PIECES)

_HAS_R = (0, 3, 4, 6)
_HAS_D = (1, 3, 5, 6)
_HAS_T = (2, 4, 5, 6)
_HAS_P = (6,)


def _vlog(x):
    bits = lax.bitcast_convert_type(x, jnp.int32)
    e = (bits >> 23) - 127
    m = lax.bitcast_convert_type((bits & 0x007FFFFF) | 0x3F800000, jnp.float32)
    big = m > _SQRT2
    m = jnp.where(big, m * 0.5, m)
    ef = e.astype(jnp.float32) + jnp.where(big, 1.0, 0.0)
    f = m - 1.0
    q = (-0.5000041083608477
         + f * (0.3332492391225158
                + f * (-0.24932832776171132
                       + f * (0.20346370495399466
                              + f * (-0.18482372758788945
                                     + f * 0.12282081708318798)))))
    return ef * _LN2 + (f + (f * f) * q)


def _member(bid, regions):
    acc = jnp.float32(0.0)
    for rid in regions:
        acc = jnp.where(bid == rid, jnp.float32(1.0), acc)
    return acc


def _sc_body(mu_r, mu_d, mu_t, lv_r, lv_d, lv_t, o_mu, o_lv, *scr):
    bufs = (scr[0:8], scr[8:16])
    bbuf = scr[16:24]
    in_sems = scr[24:26]
    out_sems = scr[26:28]
    bin_sem = scr[28]
    bout_sem = scr[29]

    wid = lax.axis_index("s") * _NC + lax.axis_index("c")
    rw = wid >> 3
    cw = wid & 7
    col = cw * _CW
    mus = (mu_r, mu_d, mu_t)
    lvs = (lv_r, lv_d, lv_t)

    gbase = [gs + jnp.minimum(rw * _GPER, (ge - gs) - _GPER)
             for (gs, ge) in _INTERIOR]

    bid = rw + 2
    brow = (bid + 1) * 584
    bhs = []
    for j in range(3):
        bhs.append(pltpu.async_copy(
            mus[j].at[pl.ds(brow, 8), pl.ds(col, _CW)], bbuf[j], bin_sem))
        bhs.append(pltpu.async_copy(
            lvs[j].at[pl.ds(brow, 8), pl.ds(col, _CW)], bbuf[3 + j], bin_sem))

    def boundary_finish():
        m0, m1, m2, l0, l1, l2, omu, olv = bbuf
        for h in bhs:
            h.wait()
        wrA = _member(bid, _HAS_R)
        wdA = _member(bid, _HAS_D)
        wtA = _member(bid, _HAS_T)
        wpA = _member(bid, _HAS_P)
        wrB = _member(bid + 1, _HAS_R)
        wdB = _member(bid + 1, _HAS_D)
        wtB = _member(bid + 1, _HAS_T)
        wpB = _member(bid + 1, _HAS_P)
        cut = bid + 1

        @plsc.parallel_loop(0, 8 * (_CW // _L), unroll=4)
        def _(i):
            r = i >> 3
            c = (i & 7) << 4
            inA = r < cut
            wr = jnp.where(inA, wrA, wrB)
            wd = jnp.where(inA, wdA, wdB)
            wt = jnp.where(inA, wtA, wtB)
            wp = jnp.where(inA, wpA, wpB)
            mua = m0[r, pl.ds(c, _L)]
            mub = m1[r, pl.ds(c, _L)]
            muc = m2[r, pl.ds(c, _L)]
            va = jnp.exp(l0[r, pl.ds(c, _L)]) + _EPS
            vb = jnp.exp(l1[r, pl.ds(c, _L)]) + _EPS
            vc = jnp.exp(l2[r, pl.ds(c, _L)]) + _EPS
            ab = va * vb
            ac = va * vc
            bc = vb * vc
            abc = ab * vc
            rec = 1.0 / (wr * bc + wd * ac + wt * ab + wp * abc)
            omu[r, pl.ds(c, _L)] = (wr * mua * bc + wd * mub * ac
                                    + wt * muc * ab) * rec
            olv[r, pl.ds(c, _L)] = _vlog(abc * rec + _EPS)

        pltpu.async_copy(omu, o_mu.at[pl.ds(brow, 8), pl.ds(col, _CW)],
                         bout_sem).wait()
        pltpu.async_copy(olv, o_lv.at[pl.ds(brow, 8), pl.ds(col, _CW)],
                         bout_sem).wait()

    def start_in(ti):
        k, off, g = _TASKS[ti]
        mods = _REGIONS[k][2]
        s = ti % 2
        row = (gbase[k] + off) * 8
        R = g * 8
        hs = []
        for j, m in enumerate(mods):
            hs.append(pltpu.async_copy(
                mus[m].at[pl.ds(row, R), pl.ds(col, _CW)],
                bufs[s][j].at[pl.ds(0, R)], in_sems[s]))
            hs.append(pltpu.async_copy(
                lvs[m].at[pl.ds(row, R), pl.ds(col, _CW)],
                bufs[s][3 + j].at[pl.ds(0, R)], in_sems[s]))
        return hs

    def start_out(ti):
        k, off, g = _TASKS[ti]
        s = ti % 2
        row = (gbase[k] + off) * 8
        R = g * 8
        return [pltpu.async_copy(bufs[s][6].at[pl.ds(0, R)],
                                 o_mu.at[pl.ds(row, R), pl.ds(col, _CW)],
                                 out_sems[s]),
                pltpu.async_copy(bufs[s][7].at[pl.ds(0, R)],
                                 o_lv.at[pl.ds(row, R), pl.ds(col, _CW)],
                                 out_sems[s])]

    def compute(ti):
        k, off, g = _TASKS[ti]
        mods = _REGIONS[k][2]
        s = ti % 2
        m0, m1, m2, l0, l1, l2, omu, olv = bufs[s]
        nvec = g * 8 * (_CW // _L)
        if len(mods) == 2:
            @plsc.parallel_loop(0, nvec, unroll=4)
            def _(i):
                r = i >> 3
                c = (i & 7) << 4
                mua = m0[r, pl.ds(c, _L)]
                mub = m1[r, pl.ds(c, _L)]
                la = l0[r, pl.ds(c, _L)]
                lb = l1[r, pl.ds(c, _L)]
                va = jnp.exp(la) + _EPS
                vb = jnp.exp(lb) + _EPS
                den = va + vb
                rec = 1.0 / den
                omu[r, pl.ds(c, _L)] = (mua * vb + mub * va) * rec
                olv[r, pl.ds(c, _L)] = (la + lb) - _vlog(den)
        else:
            @plsc.parallel_loop(0, nvec, unroll=4)
            def _(i):
                r = i >> 3
                c = (i & 7) << 4
                mua = m0[r, pl.ds(c, _L)]
                mub = m1[r, pl.ds(c, _L)]
                muc = m2[r, pl.ds(c, _L)]
                la = l0[r, pl.ds(c, _L)]
                lb = l1[r, pl.ds(c, _L)]
                lc = l2[r, pl.ds(c, _L)]
                va = jnp.exp(la) + _EPS
                vb = jnp.exp(lb) + _EPS
                vc = jnp.exp(lc) + _EPS
                ab = va * vb
                ac = va * vc
                bc = vb * vc
                abc = ab * vc
                den = ab + ac + bc + abc
                rec = 1.0 / den
                omu[r, pl.ds(c, _L)] = (mua * bc + mub * ac + muc * ab) * rec
                olv[r, pl.ds(c, _L)] = ((la + lb) + lc) - _vlog(den)

    n = len(_TASKS)
    hout = [None] * n
    hin = start_in(0)
    for i in range(n):
        nxt = None
        if i + 1 < n:
            if i >= 1:
                for h in hout[i - 1]:
                    h.wait()
            nxt = start_in(i + 1)
        for h in hin:
            h.wait()
        compute(i)
        hout[i] = start_out(i)
        hin = nxt
    boundary_finish()
    for h in hout[n - 2]:
        h.wait()
    for h in hout[n - 1]:
        h.wait()


_fused = functools.partial(
    pl.kernel,
    out_type=(jax.ShapeDtypeStruct((_B, _D), jnp.float32),
              jax.ShapeDtypeStruct((_B, _D), jnp.float32)),
    mesh=plsc.VectorSubcoreMesh(core_axis_name="c", subcore_axis_name="s",
                                num_cores=_NC, num_subcores=_NS),
    scratch_types=([pltpu.VMEM((_PMAXR, _CW), jnp.float32)] * 16
                   + [pltpu.VMEM((8, _CW), jnp.float32)] * 8
                   + [pltpu.SemaphoreType.DMA] * 6),
)(_sc_body)


@jax.jit
def kernel(mu_rgb, mu_depth, mu_touch, logvar_rgb, logvar_depth, logvar_touch):
    o_mu, o_lv = _fused(mu_rgb, mu_depth, mu_touch,
                        logvar_rgb, logvar_depth, logvar_touch)
    pre_mu = jnp.concatenate(
        [mu_rgb[:585], mu_depth[585:1170], mu_touch[1170:1752]], axis=0)
    pre_lv = jnp.concatenate(
        [logvar_rgb[:585], logvar_depth[585:1170], logvar_touch[1170:1752]],
        axis=0)
    return (lax.dynamic_update_slice(o_mu, pre_mu, (0, 0)),
            lax.dynamic_update_slice(o_lv, pre_lv, (0, 0)))
